# Initial kernel scaffold; baseline (speedup 1.0000x reference)
import jax
import jax.numpy as jnp
from jax.experimental import pallas as pl


def kernel(output, target, shard_size, target_len, origin, part, now):
    raise NotImplementedError("write your pallas kernel here")


# speedup vs baseline: 33.7357x; 33.7357x over previous
import functools

import jax
import jax.numpy as jnp
from jax import lax
from jax.experimental import pallas as pl
from jax.experimental.pallas import tpu as pltpu
from jax.experimental.pallas import tpu_sc as plsc

_CONF = 0.9


def _sc_sparse(outflat, origin, target, consts):
    (N,) = outflat.shape
    P, L = origin.shape
    (B,) = target.shape
    V = N // B
    NC, NS = 2, 16
    NW = NC * NS
    RPW = B // NW
    CH = L // 16

    mesh = plsc.VectorSubcoreMesh(core_axis_name="c", subcore_axis_name="s")

    @functools.partial(
        pl.kernel,
        out_type=(
            jax.ShapeDtypeStruct((NW, RPW), jnp.float32),
            jax.ShapeDtypeStruct((NW, RPW), jnp.float32),
            jax.ShapeDtypeStruct((NW, RPW), jnp.float32),
        ),
        mesh=mesh,
        compiler_params=pltpu.CompilerParams(needs_layout_passes=False),
        scratch_types=[
            pltpu.VMEM((L + 16,), jnp.int32),
            pltpu.VMEM((V,), jnp.int32),
            pltpu.VMEM((B + 16,), jnp.int32),
            pltpu.VMEM((32,), jnp.int32),
            pltpu.VMEM((L,), jnp.int32),
            pltpu.VMEM((L,), jnp.float32),
            pltpu.VMEM((L,), jnp.float32),
            pltpu.VMEM((RPW,), jnp.int32),
            pltpu.VMEM((RPW,), jnp.float32),
            pltpu.VMEM((RPW,), jnp.float32),
            pltpu.VMEM((RPW,), jnp.float32),
            pltpu.SemaphoreType.DMA,
        ],
    )
    def sc(outflat_h, origin_h, target_h, consts_h,
           s_h, n_h, tv_h,
           origin_v, table_v, target_v, consts_v,
           idx_v, vals_v, mask_v, tvidx_v, tvval_v, sres_v, nres_v, sem):
        w = lax.axis_index("s") * NC + lax.axis_index("c")
        p = w % P
        pltpu.sync_copy(origin_h.at[p], origin_v.at[pl.ds(0, L)])
        pltpu.sync_copy(target_h, target_v.at[pl.ds(0, B)])
        pltpu.sync_copy(consts_h, consts_v.at[pl.ds(0, 16)])
        base = consts_v[pl.ds(0, 16)][0]
        tl = consts_v[pl.ds(1 + p, 16)][0]
        lane = lax.iota(jnp.int32, 16)
        lane0 = lane == 0

        def zero_table(k, c):
            table_v[pl.ds(k * 16, 16)] = jnp.zeros((16,), jnp.int32)
            return c

        lax.fori_loop(0, V // 16, zero_table, 0)

        def scat(ch, c):
            v16 = origin_v[pl.ds(ch * 16, 16)]
            j16 = ch * 16 + lane + 1
            for l in range(16):
                plsc.store_scatter(table_v, [v16], j16, mask=lane == l)
            return c

        lax.fori_loop(0, CH, scat, 0)

        def zero_res(k, c):
            sres_v[pl.ds(k * 16, 16)] = jnp.zeros((16,), jnp.float32)
            nres_v[pl.ds(k * 16, 16)] = jnp.zeros((16,), jnp.float32)
            return c

        lax.fori_loop(0, RPW // 16, zero_res, 0)

        def row(m, c):
            i = w + NW * m
            q = i // P
            temp = base + q
            t = target_v[pl.ds(i, 16)][0]
            m16 = jnp.full((16,), m, jnp.int32)
            plsc.store_scatter(tvidx_v, [m16],
                               jnp.full((16,), i * V + t, jnp.int32),
                               mask=lane0)
            act = temp < tl - 2

            @pl.when(act)
            def _():
                iV = i * V

                def bchunk(ch, cc_):
                    v16 = origin_v[pl.ds(ch * 16, 16)]
                    j16 = ch * 16 + lane
                    lp = plsc.load_gather(table_v, [v16])
                    good = ((j16 >= temp) & (lp == j16 + 1)
                            & (v16 != 0) & (v16 != t))
                    idx_v[pl.ds(ch * 16, 16)] = v16 + iV
                    mask_v[pl.ds(ch * 16, 16)] = jnp.where(good, 1.0, 0.0)
                    return cc_

                lax.fori_loop(0, CH, bchunk, 0)
                pltpu.async_copy(outflat_h.at[idx_v], vals_v, sem).wait()

                def acc(ch, carry):
                    s_acc, n_acc = carry
                    g = vals_v[pl.ds(ch * 16, 16)]
                    mk = mask_v[pl.ds(ch * 16, 16)]
                    return s_acc + g * mk, n_acc + mk

                s_acc, n_acc = lax.fori_loop(
                    0, CH, acc,
                    (jnp.zeros((16,), jnp.float32),
                     jnp.zeros((16,), jnp.float32)))
                plsc.store_scatter(
                    sres_v, [m16],
                    jnp.full((16,), jnp.sum(s_acc), jnp.float32), mask=lane0)
                plsc.store_scatter(
                    nres_v, [m16],
                    jnp.full((16,), jnp.sum(n_acc), jnp.float32), mask=lane0)

            return c

        lax.fori_loop(0, RPW, row, 0)
        pltpu.async_copy(outflat_h.at[tvidx_v], tvval_v, sem).wait()
        pltpu.sync_copy(sres_v, s_h.at[w])
        pltpu.sync_copy(nres_v, n_h.at[w])
        pltpu.sync_copy(tvval_v, tv_h.at[w])

    return sc(outflat, origin, target, consts)


def _tc_loss(output, s3, n3, tv3, tgt3, basearr, tlen2, p_const):
    B, V = output.shape
    RB = 16
    R = B // RB
    VBLK = 6400
    VB = V // VBLK

    def body(base_r, tlen_r, x_r, s_r, n_r, tv_r, t_r, loss_r, m_scr, d_scr):
        rb = pl.program_id(0)
        vb = pl.program_id(1)
        x = x_r[...]
        bm = jnp.max(x, axis=1, keepdims=True)

        @pl.when(vb == 0)
        def _():
            m_scr[...] = jnp.broadcast_to(bm, (R, 128))
            d_scr[...] = jnp.broadcast_to(
                jnp.sum(jnp.exp(x - bm), axis=1, keepdims=True), (R, 128))

        @pl.when(vb > 0)
        def _():
            m_old = m_scr[:, :1]
            m_new = jnp.maximum(m_old, bm)
            d_new = (d_scr[:, :1] * jnp.exp(m_old - m_new)
                     + jnp.sum(jnp.exp(x - m_new), axis=1, keepdims=True))
            m_scr[...] = jnp.broadcast_to(m_new, (R, 128))
            d_scr[...] = jnp.broadcast_to(d_new, (R, 128))

        @pl.when(vb == VB - 1)
        def _():
            lse = m_scr[:, :1] + jnp.log(d_scr[:, :1])
            r = lax.broadcasted_iota(jnp.int32, (R, 1), 0)
            i = rb * R + r
            p = i % p_const
            q = i // p_const
            temp = base_r[0, 0] + q
            tl = jnp.zeros((R, 1), jnp.int32)
            for k in range(p_const):
                tl = tl + jnp.where(p == k, tlen_r[0, k], 0)
            act = temp < tl - 2
            t = t_r[0]
            s = s_r[0]
            n = n_r[0]
            tv = tv_r[0]
            wgt = jnp.where(t != 0, _CONF, 0.0)
            den = jnp.where(act, tl.astype(jnp.float32)
                            - temp.astype(jnp.float32) - 2.0, 1.0)
            csm = jnp.where(act & (t != 0), _CONF / den, 0.0)
            contrib = wgt * (tv - lse) + csm * (s - n * lse)
            part = -jnp.sum(contrib)

            @pl.when(rb == 0)
            def _():
                loss_r[0, 0] = part

            @pl.when(rb > 0)
            def _():
                loss_r[0, 0] = loss_r[0, 0] + part

    return pl.pallas_call(
        body,
        grid=(RB, VB),
        in_specs=[
            pl.BlockSpec(memory_space=pltpu.SMEM),
            pl.BlockSpec(memory_space=pltpu.SMEM),
            pl.BlockSpec((R, VBLK), lambda rb, vb: (rb, vb)),
            pl.BlockSpec((1, R, 1), lambda rb, vb: (rb, 0, 0)),
            pl.BlockSpec((1, R, 1), lambda rb, vb: (rb, 0, 0)),
            pl.BlockSpec((1, R, 1), lambda rb, vb: (rb, 0, 0)),
            pl.BlockSpec((1, R, 1), lambda rb, vb: (rb, 0, 0)),
        ],
        out_specs=pl.BlockSpec(memory_space=pltpu.SMEM),
        out_shape=jax.ShapeDtypeStruct((1, 1), jnp.float32),
        scratch_shapes=[
            pltpu.VMEM((R, 128), jnp.float32),
            pltpu.VMEM((R, 128), jnp.float32),
        ],
    )(basearr, tlen2, output, s3, n3, tv3, tgt3)


def kernel(output, target, shard_size, target_len, origin, part, now):
    B, V = output.shape
    P, L = origin.shape
    base = (jnp.asarray(now, jnp.int32)
            * jnp.asarray(shard_size, jnp.int32))
    outflat = jnp.reshape(output, (-1,))
    tgt = target.astype(jnp.int32)
    tlen = target_len.astype(jnp.int32)
    consts = jnp.concatenate(
        [base.reshape(1), tlen, jnp.zeros((16 - 1 - P,), jnp.int32)])

    s2d, n2d, tv2d = _sc_sparse(outflat, origin.astype(jnp.int32), tgt,
                                consts)
    RB = 16
    R = B // RB
    s3 = s2d.T.reshape(RB, R, 1)
    n3 = n2d.T.reshape(RB, R, 1)
    tv3 = tv2d.T.reshape(RB, R, 1)
    tgt3 = tgt.reshape(RB, R, 1)
    basearr = base.reshape(1, 1)
    tlen2 = tlen.reshape(1, P)

    loss = _tc_loss(output, s3, n3, tv3, tgt3, basearr, tlen2, P)
    return loss[0, 0]


# speedup vs baseline: 54.8640x; 1.6263x over previous
import functools

import jax
import jax.numpy as jnp
from jax import lax
from jax.experimental import pallas as pl
from jax.experimental.pallas import tpu as pltpu
from jax.experimental.pallas import tpu_sc as plsc

_CONF = 0.9


def _sc_sparse(outflat, originflat, target, consts):
    (N,) = outflat.shape
    (PL,) = originflat.shape
    (B,) = target.shape
    V = N // B
    NC, NS = 2, 16
    NW = NC * NS
    P = 8
    L = PL // P
    RPW = B // NW
    QW = RPW // P
    CH = L // 16
    NCHK = 4
    CSZ = L // NCHK
    CCH = CSZ // 16
    DEPTH = 4

    mesh = plsc.VectorSubcoreMesh(core_axis_name="c", subcore_axis_name="s")

    scratch = [
        pltpu.VMEM((PL,), jnp.int32),
        pltpu.VMEM((PL,), jnp.float32),
        pltpu.VMEM((V,), jnp.int32),
        pltpu.VMEM((B + 16,), jnp.int32),
        pltpu.VMEM((32,), jnp.int32),
        pltpu.VMEM((RPW,), jnp.int32),
        pltpu.VMEM((RPW,), jnp.float32),
        pltpu.VMEM((RPW,), jnp.float32),
        pltpu.VMEM((RPW,), jnp.float32),
    ]
    for _ in range(DEPTH):
        scratch += [pltpu.VMEM((L,), jnp.int32),
                    pltpu.VMEM((L,), jnp.float32)]
    scratch += [pltpu.SemaphoreType.DMA] * DEPTH

    @functools.partial(
        pl.kernel,
        out_type=(
            jax.ShapeDtypeStruct((NW, RPW), jnp.float32),
            jax.ShapeDtypeStruct((NW, RPW), jnp.float32),
            jax.ShapeDtypeStruct((NW, RPW), jnp.float32),
        ),
        mesh=mesh,
        compiler_params=pltpu.CompilerParams(needs_layout_passes=False),
        scratch_types=scratch,
    )
    def sc(outflat_h, origin_h, target_h, consts_h,
           s_h, n_h, tv_h,
           origin_v, islast_v, table_v, target_v, consts_v,
           tvidx_v, tvval_v, sres_v, nres_v, *bufs):
        idxbs = [bufs[2 * d] for d in range(DEPTH)]
        valbs = [bufs[2 * d + 1] for d in range(DEPTH)]
        sems = list(bufs[2 * DEPTH:])
        w = lax.axis_index("s") * NC + lax.axis_index("c")
        pltpu.sync_copy(origin_h, origin_v)
        pltpu.sync_copy(target_h, target_v.at[pl.ds(0, B)])
        pltpu.sync_copy(consts_h, consts_v.at[pl.ds(0, 16)])
        base = consts_v[pl.ds(0, 16)][0]
        lane = lax.iota(jnp.int32, 16)
        lane0 = lane == 0

        def phase0(p, c):
            pL = p * L

            def zt(k, c2):
                table_v[pl.ds(k * 16, 16)] = jnp.zeros((16,), jnp.int32)
                return c2

            lax.fori_loop(0, V // 16, zt, 0)

            def scat(ch, c2):
                v16 = origin_v[pl.ds(pL + ch * 16, 16)]
                j16 = ch * 16 + lane + 1
                for l in range(16):
                    plsc.store_scatter(table_v, [v16], j16, mask=lane == l)
                return c2

            lax.fori_loop(0, CH, scat, 0)

            def il(ch, c2):
                v16 = origin_v[pl.ds(pL + ch * 16, 16)]
                lp = plsc.load_gather(table_v, [v16])
                good = (lp == ch * 16 + lane + 1) & (v16 != 0)
                islast_v[pl.ds(pL + ch * 16, 16)] = jnp.where(good, 1.0, 0.0)
                return c2

            lax.fori_loop(0, CH, il, 0)
            return c

        lax.fori_loop(0, P, phase0, 0)

        def zero_res(k, c):
            sres_v[pl.ds(k * 16, 16)] = jnp.zeros((16,), jnp.float32)
            nres_v[pl.ds(k * 16, 16)] = jnp.zeros((16,), jnp.float32)
            return c

        lax.fori_loop(0, RPW // 16, zero_res, 0)

        def row_params(r):
            qq = r // P
            p = r % P
            q = w + NW * qq
            i = P * q + p
            temp = base + q
            tl = consts_v[pl.ds(1 + p, 16)][0]
            act = temp < tl - 2
            return p, i, temp, act

        def prep(r, idxb, valb, sem):
            p, i, temp, act = row_params(r)
            t = target_v[pl.ds(i, 16)][0]
            plsc.store_scatter(tvidx_v, [jnp.full((16,), r, jnp.int32)],
                               jnp.full((16,), i * V + t, jnp.int32),
                               mask=lane0)
            iV = i * V
            pL = p * L
            for c in range(NCHK):
                @pl.when(act & (temp < (c + 1) * CSZ))
                def _(c=c):
                    def bld(s16, c2):
                        o = c * CSZ + s16 * 16
                        idxb[pl.ds(o, 16)] = origin_v[pl.ds(pL + o, 16)] + iV
                        return c2

                    lax.fori_loop(0, CCH, bld, 0)
                    pltpu.async_copy(
                        outflat_h.at[idxb.at[pl.ds(c * CSZ, CSZ)]],
                        valb.at[pl.ds(c * CSZ, CSZ)], sem)

        def cons(r, idxb, valb, sem):
            p, i, temp, act = row_params(r)
            t = target_v[pl.ds(i, 16)][0]
            pL = p * L
            for c in range(NCHK):
                @pl.when(act & (temp < (c + 1) * CSZ))
                def _(c=c):
                    pltpu.make_async_copy(
                        outflat_h.at[idxb.at[pl.ds(c * CSZ, CSZ)]],
                        valb.at[pl.ds(c * CSZ, CSZ)], sem).wait()

            @pl.when(act)
            def _():
                def acc(ch, carry):
                    s_acc, n_acc = carry
                    o = ch * 16
                    v16 = origin_v[pl.ds(pL + o, 16)]
                    isl = islast_v[pl.ds(pL + o, 16)]
                    good = ((isl != 0.0) & (o + lane >= temp) & (v16 != t))
                    g = valb[pl.ds(o, 16)]
                    return (s_acc + jnp.where(good, g, 0.0),
                            n_acc + jnp.where(good, 1.0, 0.0))

                s_acc, n_acc = lax.fori_loop(
                    0, CH, acc,
                    (jnp.zeros((16,), jnp.float32),
                     jnp.zeros((16,), jnp.float32)))
                r16 = jnp.full((16,), r, jnp.int32)
                plsc.store_scatter(
                    sres_v, [r16],
                    jnp.full((16,), jnp.sum(s_acc), jnp.float32), mask=lane0)
                plsc.store_scatter(
                    nres_v, [r16],
                    jnp.full((16,), jnp.sum(n_acc), jnp.float32), mask=lane0)

        for b in range(DEPTH):
            prep(b, idxbs[b], valbs[b], sems[b])

        def row_loop(rr, c):
            for b in range(DEPTH):
                r = DEPTH * rr + b
                cons(r, idxbs[b], valbs[b], sems[b])

                @pl.when(r + DEPTH < RPW)
                def _(b=b, r=r):
                    prep(r + DEPTH, idxbs[b], valbs[b], sems[b])
            return c

        lax.fori_loop(0, RPW // DEPTH, row_loop, 0)

        pltpu.async_copy(outflat_h.at[tvidx_v], tvval_v, sems[0]).wait()
        pltpu.sync_copy(sres_v, s_h.at[w])
        pltpu.sync_copy(nres_v, n_h.at[w])
        pltpu.sync_copy(tvval_v, tv_h.at[w])

    return sc(outflat, originflat, target, consts)


def _tc_loss(output, s3, n3, tv3, tgt3, basearr, tlen2, p_const):
    B, V = output.shape
    RB = 16
    R = B // RB
    VBLK = 6400
    VB = V // VBLK

    def body(base_r, tlen_r, x_r, s_r, n_r, tv_r, t_r, loss_r, m_scr, d_scr):
        rb = pl.program_id(0)
        vb = pl.program_id(1)
        x = x_r[...]
        bm = jnp.max(x, axis=1, keepdims=True)

        @pl.when(vb == 0)
        def _():
            m_scr[...] = jnp.broadcast_to(bm, (R, 128))
            d_scr[...] = jnp.broadcast_to(
                jnp.sum(jnp.exp(x - bm), axis=1, keepdims=True), (R, 128))

        @pl.when(vb > 0)
        def _():
            m_old = m_scr[:, :1]
            m_new = jnp.maximum(m_old, bm)
            d_new = (d_scr[:, :1] * jnp.exp(m_old - m_new)
                     + jnp.sum(jnp.exp(x - m_new), axis=1, keepdims=True))
            m_scr[...] = jnp.broadcast_to(m_new, (R, 128))
            d_scr[...] = jnp.broadcast_to(d_new, (R, 128))

        @pl.when(vb == VB - 1)
        def _():
            lse = m_scr[:, :1] + jnp.log(d_scr[:, :1])
            r = lax.broadcasted_iota(jnp.int32, (R, 1), 0)
            i = rb * R + r
            p = i % p_const
            q = i // p_const
            temp = base_r[0, 0] + q
            tl = jnp.zeros((R, 1), jnp.int32)
            for k in range(p_const):
                tl = tl + jnp.where(p == k, tlen_r[0, k], 0)
            act = temp < tl - 2
            t = t_r[0]
            s = s_r[0]
            n = n_r[0]
            tv = tv_r[0]
            wgt = jnp.where(t != 0, _CONF, 0.0)
            den = jnp.where(act, tl.astype(jnp.float32)
                            - temp.astype(jnp.float32) - 2.0, 1.0)
            csm = jnp.where(act & (t != 0), _CONF / den, 0.0)
            contrib = wgt * (tv - lse) + csm * (s - n * lse)
            part = -jnp.sum(contrib)

            @pl.when(rb == 0)
            def _():
                loss_r[0, 0] = part

            @pl.when(rb > 0)
            def _():
                loss_r[0, 0] = loss_r[0, 0] + part

    return pl.pallas_call(
        body,
        grid=(RB, VB),
        in_specs=[
            pl.BlockSpec(memory_space=pltpu.SMEM),
            pl.BlockSpec(memory_space=pltpu.SMEM),
            pl.BlockSpec((R, VBLK), lambda rb, vb: (rb, vb)),
            pl.BlockSpec((1, R, 1), lambda rb, vb: (rb, 0, 0)),
            pl.BlockSpec((1, R, 1), lambda rb, vb: (rb, 0, 0)),
            pl.BlockSpec((1, R, 1), lambda rb, vb: (rb, 0, 0)),
            pl.BlockSpec((1, R, 1), lambda rb, vb: (rb, 0, 0)),
        ],
        out_specs=pl.BlockSpec(memory_space=pltpu.SMEM),
        out_shape=jax.ShapeDtypeStruct((1, 1), jnp.float32),
        scratch_shapes=[
            pltpu.VMEM((R, 128), jnp.float32),
            pltpu.VMEM((R, 128), jnp.float32),
        ],
    )(basearr, tlen2, output, s3, n3, tv3, tgt3)


def kernel(output, target, shard_size, target_len, origin, part, now):
    B, V = output.shape
    P, L = origin.shape
    base = (jnp.asarray(now, jnp.int32)
            * jnp.asarray(shard_size, jnp.int32))
    outflat = jnp.reshape(output, (-1,))
    tgt = target.astype(jnp.int32)
    tlen = target_len.astype(jnp.int32)
    consts = jnp.concatenate(
        [base.reshape(1), tlen, jnp.zeros((16 - 1 - P,), jnp.int32)])

    s2d, n2d, tv2d = _sc_sparse(outflat, origin.astype(jnp.int32).reshape(-1),
                                tgt, consts)
    RB = 16
    R = B // RB
    NW = 32
    def nat(x):
        return (x.reshape(NW, B // NW // P, P)
                 .transpose(1, 0, 2).reshape(B))
    s3 = nat(s2d).reshape(RB, R, 1)
    n3 = nat(n2d).reshape(RB, R, 1)
    tv3 = nat(tv2d).reshape(RB, R, 1)
    tgt3 = tgt.reshape(RB, R, 1)
    basearr = base.reshape(1, 1)
    tlen2 = tlen.reshape(1, P)

    loss = _tc_loss(output, s3, n3, tv3, tgt3, basearr, tlen2, P)
    return loss[0, 0]


# speedup vs baseline: 157.5341x; 2.8714x over previous
import functools

import jax
import jax.numpy as jnp
from jax import lax
from jax.experimental import pallas as pl
from jax.experimental.pallas import tpu as pltpu
from jax.experimental.pallas import tpu_sc as plsc

_CONF = 0.9


def _sc_sparse(output, originflat, target, consts):
    B, V = output.shape
    (PL,) = originflat.shape
    NC, NS = 2, 16
    NW = NC * NS
    P = 8
    L = PL // P
    RPW = B // NW
    CH = L // 16
    DEPTH = 2

    mesh = plsc.VectorSubcoreMesh(core_axis_name="c", subcore_axis_name="s")

    scratch = [
        pltpu.VMEM((PL,), jnp.int32),
        pltpu.VMEM((PL // 16 + 16,), jnp.int32),
        pltpu.VMEM((V,), jnp.int32),
        pltpu.VMEM((B + 16,), jnp.int32),
        pltpu.VMEM((32,), jnp.int32),
        pltpu.VMEM((RPW * 16,), jnp.float32),
        pltpu.VMEM((RPW,), jnp.float32),
        pltpu.VMEM((RPW,), jnp.float32),
        pltpu.VMEM((RPW,), jnp.float32),
    ]
    for _ in range(DEPTH):
        scratch += [pltpu.VMEM((V,), jnp.float32)]
    scratch += [pltpu.SemaphoreType.DMA] * DEPTH
    scratch += [pltpu.SemaphoreType.DMA]

    @functools.partial(
        pl.kernel,
        out_type=(
            jax.ShapeDtypeStruct((NW, RPW), jnp.float32),
            jax.ShapeDtypeStruct((NW, RPW), jnp.float32),
            jax.ShapeDtypeStruct((NW, RPW), jnp.float32),
        ),
        mesh=mesh,
        compiler_params=pltpu.CompilerParams(needs_layout_passes=False),
        scratch_types=scratch,
    )
    def sc(out_h, origin_h, target_h, consts_h,
           s_h, n_h, tv_h,
           origin_v, islast_v, table_v, target_v, consts_v,
           tvrow_v, sres_v, nres_v, tvres_v, *bufs):
        rowbufs = list(bufs[:DEPTH])
        sems = list(bufs[DEPTH:2 * DEPTH])
        tvsem = bufs[2 * DEPTH]
        w = lax.axis_index("s") * NC + lax.axis_index("c")
        pltpu.sync_copy(origin_h, origin_v)
        pltpu.sync_copy(target_h, target_v.at[pl.ds(0, B)])
        pltpu.sync_copy(consts_h, consts_v.at[pl.ds(0, 16)])
        base = consts_v[pl.ds(0, 16)][0]
        lane = lax.iota(jnp.int32, 16)
        lane0 = lane == 0

        def zt(k, c2):
            table_v[pl.ds(k * 16, 16)] = jnp.zeros((16,), jnp.int32)
            return c2

        lax.fori_loop(0, V // 16, zt, 0)

        def phase0(p, c):
            pL = p * L

            def scat(ch, c2):
                v16 = origin_v[pl.ds(pL + ch * 16, 16)]
                j16 = ch * 16 + lane + 1
                for l in range(16):
                    plsc.store_scatter(table_v, [v16], j16, mask=lane == l)
                return c2

            lax.fori_loop(0, CH, scat, 0)

            def il(ch, c2):
                v16 = origin_v[pl.ds(pL + ch * 16, 16)]
                lp = plsc.load_gather(table_v, [v16])
                good = (lp == ch * 16 + lane + 1) & (v16 != 0)
                word = jnp.sum(jnp.where(
                    good, lax.shift_left(jnp.int32(1), lane), 0))
                plsc.store_scatter(
                    islast_v, [jnp.full((16,), p * CH + ch, jnp.int32)],
                    jnp.full((16,), word, jnp.int32), mask=lane0)
                return c2

            lax.fori_loop(0, CH, il, 0)

            def clr(ch, c2):
                v16 = origin_v[pl.ds(pL + ch * 16, 16)]
                plsc.store_scatter(table_v, [v16], jnp.zeros((16,), jnp.int32))
                return c2

            lax.fori_loop(0, CH, clr, 0)
            return c

        lax.fori_loop(0, P, phase0, 0)

        def zero_res(k, c):
            sres_v[pl.ds(k * 16, 16)] = jnp.zeros((16,), jnp.float32)
            nres_v[pl.ds(k * 16, 16)] = jnp.zeros((16,), jnp.float32)
            return c

        lax.fori_loop(0, RPW // 16, zero_res, 0)

        def row_params(r):
            qq = r // P
            p = r % P
            q = w + NW * qq
            i = P * q + p
            temp = base + q
            tl = consts_v[pl.ds(1 + p, 16)][0]
            act = temp < tl - 2
            return p, i, temp, act

        def prep(r, rowbuf, sem):
            p, i, temp, act = row_params(r)
            t = target_v[pl.ds(i, 16)][0]
            talign = (t // 16) * 16
            pltpu.async_copy(out_h.at[i, pl.ds(talign, 16)],
                             tvrow_v.at[pl.ds(r * 16, 16)], tvsem)

            @pl.when(act)
            def _():
                pltpu.async_copy(out_h.at[i], rowbuf, sem)

        def cons(r, rowbuf, sem):
            p, i, temp, act = row_params(r)
            t = target_v[pl.ds(i, 16)][0]
            pL = p * L

            @pl.when(act)
            def _():
                pltpu.make_async_copy(out_h.at[i], rowbuf, sem).wait()

                def acc(ch, carry):
                    s_acc, n_acc = carry
                    o = ch * 16
                    v16 = origin_v[pl.ds(pL + o, 16)]
                    word = islast_v[pl.ds(p * CH + ch, 16)][0]
                    bit = lax.shift_right_logical(
                        jnp.full((16,), word, jnp.int32), lane) & 1
                    good = ((bit != 0) & (o + lane >= temp) & (v16 != t))
                    g = plsc.load_gather(rowbuf, [v16])
                    return (s_acc + jnp.where(good, g, 0.0),
                            n_acc + jnp.where(good, 1.0, 0.0))

                s_acc, n_acc = lax.fori_loop(
                    temp // 16, CH, acc,
                    (jnp.zeros((16,), jnp.float32),
                     jnp.zeros((16,), jnp.float32)))
                r16 = jnp.full((16,), r, jnp.int32)
                plsc.store_scatter(
                    sres_v, [r16],
                    jnp.full((16,), jnp.sum(s_acc), jnp.float32), mask=lane0)
                plsc.store_scatter(
                    nres_v, [r16],
                    jnp.full((16,), jnp.sum(n_acc), jnp.float32), mask=lane0)

        for b in range(DEPTH):
            prep(b, rowbufs[b], sems[b])

        def row_loop(rr, c):
            for b in range(DEPTH):
                r = DEPTH * rr + b
                cons(r, rowbufs[b], sems[b])

                @pl.when(r + DEPTH < RPW)
                def _(b=b, r=r):
                    prep(r + DEPTH, rowbufs[b], sems[b])
            return c

        lax.fori_loop(0, RPW // DEPTH, row_loop, 0)

        def tvfin(r, c):
            p, i, temp, act = row_params(r)
            t = target_v[pl.ds(i, 16)][0]
            talign = (t // 16) * 16
            pltpu.make_async_copy(out_h.at[i, pl.ds(talign, 16)],
                                  tvrow_v.at[pl.ds(r * 16, 16)], tvsem).wait()
            v16 = tvrow_v[pl.ds(r * 16, 16)]
            tvv = jnp.sum(jnp.where(lane == t - talign, v16, 0.0))
            plsc.store_scatter(tvres_v, [jnp.full((16,), r, jnp.int32)],
                               jnp.full((16,), tvv, jnp.float32), mask=lane0)
            return c

        lax.fori_loop(0, RPW, tvfin, 0)

        pltpu.sync_copy(sres_v, s_h.at[w])
        pltpu.sync_copy(nres_v, n_h.at[w])
        pltpu.sync_copy(tvres_v, tv_h.at[w])

    return sc(output, originflat, target, consts)


def _tc_lse(output):
    B, V = output.shape
    RB = 16
    R = B // RB
    VBLK = 6400
    VB = V // VBLK

    def body(x_r, lse_r, m_scr, d_scr):
        vb = pl.program_id(1)
        x = x_r[...]
        bm = jnp.max(x, axis=1, keepdims=True)

        @pl.when(vb == 0)
        def _():
            m_scr[...] = jnp.broadcast_to(bm, (R, 128))
            d_scr[...] = jnp.broadcast_to(
                jnp.sum(jnp.exp(x - bm), axis=1, keepdims=True), (R, 128))

        @pl.when(vb > 0)
        def _():
            m_old = m_scr[:, :1]
            m_new = jnp.maximum(m_old, bm)
            d_new = (d_scr[:, :1] * jnp.exp(m_old - m_new)
                     + jnp.sum(jnp.exp(x - m_new), axis=1, keepdims=True))
            m_scr[...] = jnp.broadcast_to(m_new, (R, 128))
            d_scr[...] = jnp.broadcast_to(d_new, (R, 128))

        @pl.when(vb == VB - 1)
        def _():
            lse_r[0] = m_scr[:, :1] + jnp.log(d_scr[:, :1])

    return pl.pallas_call(
        body,
        grid=(RB, VB),
        in_specs=[pl.BlockSpec((R, VBLK), lambda rb, vb: (rb, vb))],
        out_specs=pl.BlockSpec((1, R, 1), lambda rb, vb: (rb, 0, 0)),
        out_shape=jax.ShapeDtypeStruct((RB, R, 1), jnp.float32),
        scratch_shapes=[
            pltpu.VMEM((R, 128), jnp.float32),
            pltpu.VMEM((R, 128), jnp.float32),
        ],
    )(output)


def _tc_combine(lse2, s2, n2, tv2, tgt2, basearr, tlen2, p_const):
    RB, R = lse2.shape

    def body(base_r, tlen_r, lse_r, s_r, n_r, tv_r, t_r, loss_r):
        d0 = lax.broadcasted_iota(jnp.int32, (RB, R), 0)
        d1 = lax.broadcasted_iota(jnp.int32, (RB, R), 1)
        i = d0 * R + d1
        p = i % p_const
        q = i // p_const
        temp = base_r[0, 0] + q
        tl = jnp.zeros((RB, R), jnp.int32)
        for k in range(p_const):
            tl = tl + jnp.where(p == k, tlen_r[0, k], 0)
        act = temp < tl - 2
        t = t_r[...]
        lse = lse_r[...]
        wgt = jnp.where(t != 0, _CONF, 0.0)
        den = jnp.where(act, tl.astype(jnp.float32)
                        - temp.astype(jnp.float32) - 2.0, 1.0)
        csm = jnp.where(act & (t != 0), _CONF / den, 0.0)
        contrib = wgt * (tv_r[...] - lse) + csm * (s_r[...] - n_r[...] * lse)
        loss_r[0, 0] = -jnp.sum(contrib)

    return pl.pallas_call(
        body,
        in_specs=[
            pl.BlockSpec(memory_space=pltpu.SMEM),
            pl.BlockSpec(memory_space=pltpu.SMEM),
            pl.BlockSpec((RB, R), lambda: (0, 0)),
            pl.BlockSpec((RB, R), lambda: (0, 0)),
            pl.BlockSpec((RB, R), lambda: (0, 0)),
            pl.BlockSpec((RB, R), lambda: (0, 0)),
            pl.BlockSpec((RB, R), lambda: (0, 0)),
        ],
        out_specs=pl.BlockSpec(memory_space=pltpu.SMEM),
        out_shape=jax.ShapeDtypeStruct((1, 1), jnp.float32),
    )(basearr, tlen2, lse2, s2, n2, tv2, tgt2)


def kernel(output, target, shard_size, target_len, origin, part, now):
    B, V = output.shape
    P, L = origin.shape
    base = (jnp.asarray(now, jnp.int32)
            * jnp.asarray(shard_size, jnp.int32))
    tgt = target.astype(jnp.int32)
    tlen = target_len.astype(jnp.int32)
    consts = jnp.concatenate(
        [base.reshape(1), tlen, jnp.zeros((16 - 1 - P,), jnp.int32)])

    s2d, n2d, tv2d = _sc_sparse(output, origin.astype(jnp.int32).reshape(-1),
                                tgt, consts)
    lse3 = _tc_lse(output)
    RB = 16
    R = B // RB
    NW = 32

    def nat(x):
        return (x.reshape(NW, B // NW // P, P)
                 .transpose(1, 0, 2).reshape(RB, R))

    loss = _tc_combine(lse3.reshape(RB, R), nat(s2d), nat(n2d), nat(tv2d),
                       tgt.reshape(RB, R), base.reshape(1, 1),
                       tlen.reshape(1, P), P)
    return loss[0, 0]


# speedup vs baseline: 168.2706x; 1.0682x over previous
import functools

import jax
import jax.numpy as jnp
from jax import lax
from jax.experimental import pallas as pl
from jax.experimental.pallas import tpu as pltpu
from jax.experimental.pallas import tpu_sc as plsc

_CONF = 0.9


def _sc_sparse(output, originflat, target, consts):
    B, V = output.shape
    (PL,) = originflat.shape
    NC, NS = 2, 16
    NW = NC * NS
    P = 8
    L = PL // P
    RPW = B // NW
    CH = L // 16
    DEPTH = 2

    mesh = plsc.VectorSubcoreMesh(core_axis_name="c", subcore_axis_name="s")

    scratch = [
        pltpu.VMEM((PL,), jnp.int32),
        pltpu.VMEM((PL // 16 + 16,), jnp.int32),
        pltpu.VMEM((V,), jnp.int32),
        pltpu.VMEM((B + 16,), jnp.int32),
        pltpu.VMEM((32,), jnp.int32),
        pltpu.VMEM((RPW * 16,), jnp.float32),
        pltpu.VMEM((RPW,), jnp.float32),
        pltpu.VMEM((RPW,), jnp.float32),
        pltpu.VMEM((RPW,), jnp.float32),
    ]
    for _ in range(DEPTH):
        scratch += [pltpu.VMEM((V,), jnp.float32)]
    scratch += [pltpu.SemaphoreType.DMA] * DEPTH
    scratch += [pltpu.SemaphoreType.DMA]

    @functools.partial(
        pl.kernel,
        out_type=(
            jax.ShapeDtypeStruct((NW, RPW), jnp.float32),
            jax.ShapeDtypeStruct((NW, RPW), jnp.float32),
            jax.ShapeDtypeStruct((NW, RPW), jnp.float32),
        ),
        mesh=mesh,
        compiler_params=pltpu.CompilerParams(needs_layout_passes=False),
        scratch_types=scratch,
    )
    def sc(out_h, origin_h, target_h, consts_h,
           s_h, n_h, tv_h,
           origin_v, islast_v, table_v, target_v, consts_v,
           tvrow_v, sres_v, nres_v, tvres_v, *bufs):
        rowbufs = list(bufs[:DEPTH])
        sems = list(bufs[DEPTH:2 * DEPTH])
        tvsem = bufs[2 * DEPTH]
        w = lax.axis_index("s") * NC + lax.axis_index("c")
        pltpu.sync_copy(origin_h, origin_v)
        pltpu.sync_copy(target_h, target_v.at[pl.ds(0, B)])
        pltpu.sync_copy(consts_h, consts_v.at[pl.ds(0, 16)])
        base = consts_v[pl.ds(0, 16)][0]
        lane = lax.iota(jnp.int32, 16)
        lane0 = lane == 0

        def zt(k, c2):
            table_v[pl.ds(k * 16, 16)] = jnp.zeros((16,), jnp.int32)
            return c2

        lax.fori_loop(0, V // 16, zt, 0)

        def phase0(p, c):
            pL = p * L

            def scat(ch, c2):
                v16 = origin_v[pl.ds(pL + ch * 16, 16)]
                j16 = ch * 16 + lane + 1
                for l in range(16):
                    plsc.store_scatter(table_v, [v16], j16, mask=lane == l)
                return c2

            lax.fori_loop(0, CH, scat, 0)

            def il(ch, c2):
                v16 = origin_v[pl.ds(pL + ch * 16, 16)]
                lp = plsc.load_gather(table_v, [v16])
                good = (lp == ch * 16 + lane + 1) & (v16 != 0)
                word = jnp.sum(jnp.where(
                    good, lax.shift_left(jnp.int32(1), lane), 0))
                plsc.store_scatter(
                    islast_v, [jnp.full((16,), p * CH + ch, jnp.int32)],
                    jnp.full((16,), word, jnp.int32), mask=lane0)
                return c2

            lax.fori_loop(0, CH, il, 0)

            def clr(ch, c2):
                v16 = origin_v[pl.ds(pL + ch * 16, 16)]
                plsc.store_scatter(table_v, [v16], jnp.zeros((16,), jnp.int32))
                return c2

            lax.fori_loop(0, CH, clr, 0)
            return c

        lax.fori_loop(0, P, phase0, 0)

        def zero_res(k, c):
            sres_v[pl.ds(k * 16, 16)] = jnp.zeros((16,), jnp.float32)
            nres_v[pl.ds(k * 16, 16)] = jnp.zeros((16,), jnp.float32)
            return c

        lax.fori_loop(0, RPW // 16, zero_res, 0)

        def row_params(r):
            qq = r // P
            p = r % P
            q = w + NW * qq
            i = P * q + p
            temp = base + q
            tl = consts_v[pl.ds(1 + p, 16)][0]
            act = temp < tl - 2
            return p, i, temp, act

        def prep(r, rowbuf, sem):
            p, i, temp, act = row_params(r)
            t = target_v[pl.ds(i, 16)][0]
            talign = (t // 16) * 16
            pltpu.async_copy(out_h.at[i, pl.ds(talign, 16)],
                             tvrow_v.at[pl.ds(r * 16, 16)], tvsem)

            @pl.when(act)
            def _():
                pltpu.async_copy(out_h.at[i], rowbuf, sem)

        def cons(r, rowbuf, sem):
            p, i, temp, act = row_params(r)
            t = target_v[pl.ds(i, 16)][0]
            pL = p * L

            @pl.when(act)
            def _():
                pltpu.make_async_copy(out_h.at[i], rowbuf, sem).wait()

                def acc(ch, carry):
                    s_acc, n_acc = carry
                    o = ch * 16
                    v16 = origin_v[pl.ds(pL + o, 16)]
                    word = islast_v[pl.ds(p * CH + ch, 16)][0]
                    bit = lax.shift_right_logical(
                        jnp.full((16,), word, jnp.int32), lane) & 1
                    good = ((bit != 0) & (o + lane >= temp) & (v16 != t))
                    g = plsc.load_gather(rowbuf, [v16])
                    return (s_acc + jnp.where(good, g, 0.0),
                            n_acc + jnp.where(good, 1.0, 0.0))

                s_acc, n_acc = lax.fori_loop(
                    temp // 16, CH, acc,
                    (jnp.zeros((16,), jnp.float32),
                     jnp.zeros((16,), jnp.float32)))
                r16 = jnp.full((16,), r, jnp.int32)
                plsc.store_scatter(
                    sres_v, [r16],
                    jnp.full((16,), jnp.sum(s_acc), jnp.float32), mask=lane0)
                plsc.store_scatter(
                    nres_v, [r16],
                    jnp.full((16,), jnp.sum(n_acc), jnp.float32), mask=lane0)

        for b in range(DEPTH):
            prep(b, rowbufs[b], sems[b])

        def row_loop(rr, c):
            for b in range(DEPTH):
                r = DEPTH * rr + b
                cons(r, rowbufs[b], sems[b])

                @pl.when(r + DEPTH < RPW)
                def _(b=b, r=r):
                    prep(r + DEPTH, rowbufs[b], sems[b])
            return c

        lax.fori_loop(0, RPW // DEPTH, row_loop, 0)

        def tvfin(r, c):
            p, i, temp, act = row_params(r)
            t = target_v[pl.ds(i, 16)][0]
            talign = (t // 16) * 16
            pltpu.make_async_copy(out_h.at[i, pl.ds(talign, 16)],
                                  tvrow_v.at[pl.ds(r * 16, 16)], tvsem).wait()
            v16 = tvrow_v[pl.ds(r * 16, 16)]
            tvv = jnp.sum(jnp.where(lane == t - talign, v16, 0.0))
            plsc.store_scatter(tvres_v, [jnp.full((16,), r, jnp.int32)],
                               jnp.full((16,), tvv, jnp.float32), mask=lane0)
            return c

        lax.fori_loop(0, RPW, tvfin, 0)

        pltpu.sync_copy(sres_v, s_h.at[w])
        pltpu.sync_copy(nres_v, n_h.at[w])
        pltpu.sync_copy(tvres_v, tv_h.at[w])

    return sc(output, originflat, target, consts)


def _tc_lse(output):
    B, V = output.shape
    R = 128
    RB = B // R

    def body(x_r, lse_r):
        x = x_r[...]
        m = jnp.max(x, axis=1, keepdims=True)
        d = jnp.sum(jnp.exp(x - m), axis=1, keepdims=True)
        lse_r[0] = m + jnp.log(d)

    return pl.pallas_call(
        body,
        grid=(RB,),
        in_specs=[pl.BlockSpec((R, V), lambda rb: (rb, 0))],
        out_specs=pl.BlockSpec((1, R, 1), lambda rb: (rb, 0, 0)),
        out_shape=jax.ShapeDtypeStruct((RB, R, 1), jnp.float32),
        compiler_params=pltpu.CompilerParams(
            vmem_limit_bytes=100 * 1024 * 1024),
    )(output)


def _tc_combine(lse2, s2, n2, tv2, tgt2, basearr, tlen2, p_const):
    RB, R = lse2.shape

    def body(base_r, tlen_r, lse_r, s_r, n_r, tv_r, t_r, loss_r):
        d0 = lax.broadcasted_iota(jnp.int32, (RB, R), 0)
        d1 = lax.broadcasted_iota(jnp.int32, (RB, R), 1)
        i = d0 * R + d1
        p = i % p_const
        q = i // p_const
        temp = base_r[0, 0] + q
        tl = jnp.zeros((RB, R), jnp.int32)
        for k in range(p_const):
            tl = tl + jnp.where(p == k, tlen_r[0, k], 0)
        act = temp < tl - 2
        t = t_r[...]
        lse = lse_r[...]
        wgt = jnp.where(t != 0, _CONF, 0.0)
        den = jnp.where(act, tl.astype(jnp.float32)
                        - temp.astype(jnp.float32) - 2.0, 1.0)
        csm = jnp.where(act & (t != 0), _CONF / den, 0.0)
        contrib = wgt * (tv_r[...] - lse) + csm * (s_r[...] - n_r[...] * lse)
        loss_r[0, 0] = -jnp.sum(contrib)

    return pl.pallas_call(
        body,
        in_specs=[
            pl.BlockSpec(memory_space=pltpu.SMEM),
            pl.BlockSpec(memory_space=pltpu.SMEM),
            pl.BlockSpec((RB, R), lambda: (0, 0)),
            pl.BlockSpec((RB, R), lambda: (0, 0)),
            pl.BlockSpec((RB, R), lambda: (0, 0)),
            pl.BlockSpec((RB, R), lambda: (0, 0)),
            pl.BlockSpec((RB, R), lambda: (0, 0)),
        ],
        out_specs=pl.BlockSpec(memory_space=pltpu.SMEM),
        out_shape=jax.ShapeDtypeStruct((1, 1), jnp.float32),
    )(basearr, tlen2, lse2, s2, n2, tv2, tgt2)


def kernel(output, target, shard_size, target_len, origin, part, now):
    B, V = output.shape
    P, L = origin.shape
    base = (jnp.asarray(now, jnp.int32)
            * jnp.asarray(shard_size, jnp.int32))
    tgt = target.astype(jnp.int32)
    tlen = target_len.astype(jnp.int32)
    consts = jnp.concatenate(
        [base.reshape(1), tlen, jnp.zeros((16 - 1 - P,), jnp.int32)])

    s2d, n2d, tv2d = _sc_sparse(output, origin.astype(jnp.int32).reshape(-1),
                                tgt, consts)
    lse3 = _tc_lse(output)
    RB = 16
    R = B // RB
    NW = 32

    def nat(x):
        return (x.reshape(NW, B // NW // P, P)
                 .transpose(1, 0, 2).reshape(RB, R))

    loss = _tc_combine(lse3.reshape(RB, R), nat(s2d), nat(n2d), nat(tv2d),
                       tgt.reshape(RB, R), base.reshape(1, 1),
                       tlen.reshape(1, P), P)
    return loss[0, 0]
